# baseline (speedup 1.0000x reference)
import jax
import jax.numpy as jnp
from jax import lax
from jax.experimental import pallas as pl
from jax.experimental.pallas import tpu as pltpu
from jax.experimental.pallas import tpu_sc as plsc

EMBED_DIM = 64
BATCH = 16384
NUM_TABLES = 8
NC, NS = 2, 16
NW = NC * NS
B_PER_W = BATCH // NW
CHUNK = 128
NCHUNK = B_PER_W // CHUNK


def _body(*refs):
    idx_refs = refs[:NUM_TABLES]
    table_refs = refs[NUM_TABLES:2 * NUM_TABLES]
    out_refs = refs[2 * NUM_TABLES:3 * NUM_TABLES]
    idx_v, rows_v, sem = refs[3 * NUM_TABLES:]

    wid = lax.axis_index("s") * NC + lax.axis_index("c")
    base = wid * B_PER_W

    for t in range(NUM_TABLES):
        pltpu.sync_copy(idx_refs[t].at[wid], idx_v)
        for j in range(NCHUNK):
            pltpu.async_copy(
                table_refs[t].at[idx_v.at[j]],
                rows_v.at[pl.ds(j * CHUNK, CHUNK)],
                sem,
            )
        for j in range(NCHUNK):
            pltpu.make_async_copy(
                table_refs[t].at[idx_v.at[j]],
                rows_v.at[pl.ds(j * CHUNK, CHUNK)],
                sem,
            ).wait()
        pltpu.sync_copy(rows_v, out_refs[t].at[pl.ds(base, B_PER_W)])


@jax.jit
def _gather_all(*args):
    idxs = tuple(a.reshape(NW, NCHUNK, CHUNK) for a in args[:NUM_TABLES])
    tables = args[NUM_TABLES:]
    mesh = plsc.VectorSubcoreMesh(
        core_axis_name="c", subcore_axis_name="s",
        num_cores=NC, num_subcores=NS)
    out_type = tuple(
        jax.ShapeDtypeStruct((BATCH, EMBED_DIM), jnp.float32)
        for _ in range(NUM_TABLES))
    return pl.kernel(
        _body,
        out_type=out_type,
        mesh=mesh,
        compiler_params=pltpu.CompilerParams(use_tc_tiling_on_sc=False),
        scratch_types=[
            pltpu.VMEM((NCHUNK, CHUNK), jnp.int32),
            pltpu.VMEM((B_PER_W, EMBED_DIM), jnp.float32),
            pltpu.SemaphoreType.DMA,
        ],
    )(*idxs, *tables)


def kernel(pat_idx, vis_idx, symp_idx, proc_idx, dis_idx, med_idx, anat_idx,
           pharma_idx, pat_table, vis_table, symp_table, proc_table,
           dis_table, med_table, anat_table, pharma_table):
    outs = _gather_all(
        pat_idx, vis_idx, symp_idx, proc_idx, dis_idx, med_idx, anat_idx,
        pharma_idx, pat_table, vis_table, symp_table, proc_table,
        dis_table, med_table, anat_table, pharma_table)
    x_pat, x_vis, x_symp, x_proc, x_dis, x_med, x_anat, x_pharma = outs
    return (x_pat, x_vis, x_proc, x_dis, x_med, x_symp, x_anat, x_pharma)


# speedup vs baseline: 1.0067x; 1.0067x over previous
import jax
import jax.numpy as jnp
from jax import lax
from jax.experimental import pallas as pl
from jax.experimental.pallas import tpu as pltpu
from jax.experimental.pallas import tpu_sc as plsc

EMBED_DIM = 64
BATCH = 16384
NUM_TABLES = 8
NC, NS = 2, 16
NW = NC * NS
B_PER_W = BATCH // NW
CHUNK = 128
NCHUNK = B_PER_W // CHUNK
NBUF = 3


def _body(*refs):
    idx_refs = refs[:NUM_TABLES]
    table_refs = refs[NUM_TABLES:2 * NUM_TABLES]
    out_refs = refs[2 * NUM_TABLES:3 * NUM_TABLES]
    rest = refs[3 * NUM_TABLES:]
    idx_v = rest[0]
    rows = rest[1:1 + NBUF]
    sem_i = rest[1 + NBUF]
    sem_g = rest[2 + NBUF:2 + 2 * NBUF]
    sem_s = rest[2 + 2 * NBUF:2 + 3 * NBUF]

    wid = lax.axis_index("s") * NC + lax.axis_index("c")
    base = wid * B_PER_W

    for t in range(NUM_TABLES):
        pltpu.async_copy(idx_refs[t].at[wid], idx_v.at[t], sem_i)
    for t in range(NUM_TABLES):
        pltpu.make_async_copy(idx_refs[t].at[wid], idx_v.at[t], sem_i).wait()

    def issue_gathers(t):
        b = t % NBUF
        for j in range(NCHUNK):
            pltpu.async_copy(
                table_refs[t].at[idx_v.at[t].at[j]],
                rows[b].at[pl.ds(j * CHUNK, CHUNK)],
                sem_g[b],
            )

    def wait_gathers(t):
        b = t % NBUF
        for j in range(NCHUNK):
            pltpu.make_async_copy(
                table_refs[t].at[idx_v.at[t].at[j]],
                rows[b].at[pl.ds(j * CHUNK, CHUNK)],
                sem_g[b],
            ).wait()

    def issue_store(t):
        b = t % NBUF
        pltpu.async_copy(rows[b], out_refs[t].at[pl.ds(base, B_PER_W)], sem_s[b])

    def wait_store(t):
        b = t % NBUF
        pltpu.make_async_copy(
            rows[b], out_refs[t].at[pl.ds(base, B_PER_W)], sem_s[b]).wait()

    for t in range(min(NBUF, NUM_TABLES)):
        issue_gathers(t)
    for t in range(NUM_TABLES):
        wait_gathers(t)
        issue_store(t)
        nxt = t + NBUF
        if nxt < NUM_TABLES:
            wait_store(nxt - NBUF)
            issue_gathers(nxt)
    for t in range(NUM_TABLES - NBUF, NUM_TABLES):
        wait_store(t)


@jax.jit
def _gather_all(*args):
    idxs = tuple(a.reshape(NW, NCHUNK, CHUNK) for a in args[:NUM_TABLES])
    tables = args[NUM_TABLES:]
    mesh = plsc.VectorSubcoreMesh(
        core_axis_name="c", subcore_axis_name="s",
        num_cores=NC, num_subcores=NS)
    out_type = tuple(
        jax.ShapeDtypeStruct((BATCH, EMBED_DIM), jnp.float32)
        for _ in range(NUM_TABLES))
    scratch = [pltpu.VMEM((NUM_TABLES, NCHUNK, CHUNK), jnp.int32)]
    scratch += [pltpu.VMEM((B_PER_W, EMBED_DIM), jnp.float32)
                for _ in range(NBUF)]
    scratch += [pltpu.SemaphoreType.DMA for _ in range(1 + 2 * NBUF)]
    return pl.kernel(
        _body,
        out_type=out_type,
        mesh=mesh,
        compiler_params=pltpu.CompilerParams(use_tc_tiling_on_sc=False),
        scratch_types=scratch,
    )(*idxs, *tables)


def kernel(pat_idx, vis_idx, symp_idx, proc_idx, dis_idx, med_idx, anat_idx,
           pharma_idx, pat_table, vis_table, symp_table, proc_table,
           dis_table, med_table, anat_table, pharma_table):
    outs = _gather_all(
        pat_idx, vis_idx, symp_idx, proc_idx, dis_idx, med_idx, anat_idx,
        pharma_idx, pat_table, vis_table, symp_table, proc_table,
        dis_table, med_table, anat_table, pharma_table)
    x_pat, x_vis, x_symp, x_proc, x_dis, x_med, x_anat, x_pharma = outs
    return (x_pat, x_vis, x_proc, x_dis, x_med, x_symp, x_anat, x_pharma)


# speedup vs baseline: 1.0104x; 1.0037x over previous
import jax
import jax.numpy as jnp
from jax import lax
from jax.experimental import pallas as pl
from jax.experimental.pallas import tpu as pltpu
from jax.experimental.pallas import tpu_sc as plsc

EMBED_DIM = 64
BATCH = 16384
NUM_TABLES = 8
NC, NS = 2, 16
NW = NC * NS
B_PER_W = BATCH // NW
CHUNK = 128
NCHUNK = B_PER_W // CHUNK
NBUF = 3


def _body(*refs):
    idx_refs = refs[:NUM_TABLES]
    table_refs = refs[NUM_TABLES:2 * NUM_TABLES]
    out_refs = refs[2 * NUM_TABLES:3 * NUM_TABLES]
    rest = refs[3 * NUM_TABLES:]
    idx_v = rest[0]
    rows = rest[1:1 + NBUF]
    sem_i = rest[1 + NBUF]
    sem_g = rest[2 + NBUF:2 + 2 * NBUF]
    sem_s = rest[2 + 2 * NBUF:2 + 3 * NBUF]

    wid = lax.axis_index("s") * NC + lax.axis_index("c")
    base = wid * B_PER_W

    for t in range(NUM_TABLES):
        pltpu.async_copy(idx_refs[t].at[wid], idx_v.at[t], sem_i)
    for t in range(NUM_TABLES):
        pltpu.make_async_copy(idx_refs[t].at[wid], idx_v.at[t], sem_i).wait()

    def issue_gathers(t):
        b = t % NBUF
        for j in range(NCHUNK):
            pltpu.async_copy(
                table_refs[t].at[idx_v.at[t].at[j]],
                rows[b].at[pl.ds(j * CHUNK, CHUNK)],
                sem_g[b],
            )

    def wait_gathers(t):
        b = t % NBUF
        for j in range(NCHUNK):
            pltpu.make_async_copy(
                table_refs[t].at[idx_v.at[t].at[j]],
                rows[b].at[pl.ds(j * CHUNK, CHUNK)],
                sem_g[b],
            ).wait()

    def issue_store(t):
        b = t % NBUF
        pltpu.async_copy(rows[b], out_refs[t].at[pl.ds(base, B_PER_W)], sem_s[b])

    def wait_store(t):
        b = t % NBUF
        pltpu.make_async_copy(
            rows[b], out_refs[t].at[pl.ds(base, B_PER_W)], sem_s[b]).wait()

    for t in range(min(NBUF, NUM_TABLES)):
        issue_gathers(t)
    for t in range(NUM_TABLES):
        wait_gathers(t)
        issue_store(t)
        nxt = t + NBUF
        if nxt < NUM_TABLES:
            wait_store(nxt - NBUF)
            issue_gathers(nxt)
    for t in range(NUM_TABLES - NBUF, NUM_TABLES):
        wait_store(t)


@jax.jit
def _gather_all(*args):
    idxs = tuple(a.reshape(NW, NCHUNK, CHUNK) for a in args[:NUM_TABLES])
    flats = lax.optimization_barrier(
        tuple(t.reshape(-1) for t in args[NUM_TABLES:]))
    tables = tuple(f.reshape(t.shape)
                   for f, t in zip(flats, args[NUM_TABLES:]))
    mesh = plsc.VectorSubcoreMesh(
        core_axis_name="c", subcore_axis_name="s",
        num_cores=NC, num_subcores=NS)
    out_type = tuple(
        jax.ShapeDtypeStruct((BATCH, EMBED_DIM), jnp.float32)
        for _ in range(NUM_TABLES))
    scratch = [pltpu.VMEM((NUM_TABLES, NCHUNK, CHUNK), jnp.int32)]
    scratch += [pltpu.VMEM((B_PER_W, EMBED_DIM), jnp.float32)
                for _ in range(NBUF)]
    scratch += [pltpu.SemaphoreType.DMA for _ in range(1 + 2 * NBUF)]
    return pl.kernel(
        _body,
        out_type=out_type,
        mesh=mesh,
        compiler_params=pltpu.CompilerParams(use_tc_tiling_on_sc=False),
        scratch_types=scratch,
    )(*idxs, *tables)


def kernel(pat_idx, vis_idx, symp_idx, proc_idx, dis_idx, med_idx, anat_idx,
           pharma_idx, pat_table, vis_table, symp_table, proc_table,
           dis_table, med_table, anat_table, pharma_table):
    outs = _gather_all(
        pat_idx, vis_idx, symp_idx, proc_idx, dis_idx, med_idx, anat_idx,
        pharma_idx, pat_table, vis_table, symp_table, proc_table,
        dis_table, med_table, anat_table, pharma_table)
    x_pat, x_vis, x_symp, x_proc, x_dis, x_med, x_anat, x_pharma = outs
    return (x_pat, x_vis, x_proc, x_dis, x_med, x_symp, x_anat, x_pharma)
